# baseline (speedup 1.0000x reference)
import functools

import jax
import jax.numpy as jnp
from jax import lax
from jax.experimental import pallas as pl
from jax.experimental.pallas import tpu as pltpu
from jax.experimental.pallas import tpu_sc as plsc

_N = 2048
_D = 512
_H = 256
_S = 128
_P = 256
_E = 65536
_T = 3 * _N
_M = _T + _P
_RB = 256
_NBLK = _M // _RB

_NC, _NS = 2, 16
_NW = _NC * _NS
_EROWS = 3 * _E // 128
_WROWS = _EROWS // _NW
_RROWS = _T // _NW
_BROWS = _P // _NW

_PREC = jax.lax.Precision.HIGHEST


def _mlp(x, W1, b1, W2, b2):
    h = jnp.maximum(jnp.dot(x, W1.T, precision=_PREC) + b1, 0.0)
    return jnp.dot(h, W2.T, precision=_PREC) + b2


def _encode_body(xt, xv, xs, sp,
                 W1t, b1t, W2t, b2t,
                 W1v, b1v, W2v, b2v,
                 W1s, b1s, W2s, b2s,
                 W1u, b1u, W2u, b2u,
                 e0t, e1t, e0v, e1v, e0s, e1s,
                 f2_ref, ef_ref):
    f2_ref[0:_N, :] = _mlp(xt[...], W1t[...], b1t[...], W2t[...], b2t[...])
    f2_ref[_N:2 * _N, :] = _mlp(xv[...], W1v[...], b1v[...], W2v[...], b2v[...])
    f2_ref[2 * _N:_T, :] = _mlp(xs[...], W1s[...], b1s[...], W2s[...], b2s[...])
    f2_ref[_T:_M, :] = _mlp(sp[...], W1u[...], b1u[...], W2u[...], b2u[...])
    r = _E // 128
    ef_ref[0:r, :] = e0t[...] * _M + e1t[...]
    ef_ref[r:2 * r, :] = e0v[...] * _M + e1v[...] + _N * (_M + 1)
    ef_ref[2 * r:3 * r, :] = e0s[...] * _M + e1s[...] + 2 * _N * (_M + 1)


def _strips_body(f2_ref, rl_ref, rr_ref, bot_ref):
    f2 = f2_ref[...]
    sn = f2[_T:, :]
    feats = f2[:_T, :]
    right = jax.nn.sigmoid(
        lax.dot_general(feats, sn, (((1,), (1,)), ((), ())), precision=_PREC))
    rl_ref[...] = right[:, :128]
    rr_ref[...] = right[:, 128:]
    bot_ref[...] = jax.nn.sigmoid(
        lax.dot_general(sn, f2, (((1,), (1,)), ((), ())), precision=_PREC))


def _relayout_body(in_ref, out_ref):
    out_ref[...] = in_ref[...].reshape(_RB, _M)


def _paint_body(out_ref):
    out_ref[...] = jnp.zeros((_M * _M // _NBLK // 128, 128), jnp.float32)


@functools.cache
def _get_scatter():
    mesh = plsc.VectorSubcoreMesh(
        core_axis_name="c", subcore_axis_name="s",
        num_cores=_NC, num_subcores=_NS)

    @functools.partial(
        pl.kernel,
        out_type=(),
        mesh=mesh,
        scratch_types=[
            pltpu.VMEM((_WROWS, 128), jnp.int32),
            pltpu.VMEM((128,), jnp.float32),
            pltpu.VMEM((_RROWS, 256), jnp.float32),
            pltpu.VMEM((_BROWS * _M,), jnp.float32),
            pltpu.SemaphoreType.DMA,
            pltpu.SemaphoreType.DMA,
        ],
    )
    def _scatter(ef_hbm, ones_hbm, rl_hbm, rr_hbm, bot_hbm, out_ref,
                 idx_v, ones_v, right_v, bot_v, sem, sem2):
        wid = lax.axis_index("s") * _NC + lax.axis_index("c")
        base = wid * _WROWS
        pltpu.sync_copy(ef_hbm.at[pl.ds(base, _WROWS)], idx_v)
        pltpu.sync_copy(ones_hbm, ones_v)
        descs = [
            pltpu.async_copy(ones_v, out_ref.at[idx_v.at[j]], sem)
            for j in range(_WROWS)
        ]
        rbase = wid * _RROWS
        pltpu.sync_copy(rl_hbm.at[pl.ds(rbase, _RROWS)], right_v.at[:, pl.ds(0, 128)])
        pltpu.sync_copy(rr_hbm.at[pl.ds(rbase, _RROWS)], right_v.at[:, pl.ds(128, 128)])

        def _fire(k, _):
            pltpu.async_copy(
                right_v.at[k],
                out_ref.at[pl.ds((rbase + k) * _M + _T, 256)],
                sem2)
            return _
        lax.fori_loop(0, _RROWS, _fire, 0)
        bbase = _T * _M + wid * _BROWS * _M
        pltpu.sync_copy(bot_hbm.at[pl.ds(wid * _BROWS * _M, _BROWS * _M)], bot_v)
        pltpu.sync_copy(bot_v, out_ref.at[pl.ds(bbase, _BROWS * _M)])

        def _drain(k, _):
            pltpu.make_async_copy(
                right_v.at[0],
                out_ref.at[pl.ds(_T, 256)],
                sem2).wait()
            return _
        lax.fori_loop(0, _RROWS, _drain, 0)
        for d in descs:
            d.wait()

    return _scatter


def kernel(x_text, x_vision, x_structure, ei_text, ei_vision, ei_structure,
           W1_text, b1_text, W2_text, b2_text,
           W1_vision, b1_vision, W2_vision, b2_vision,
           W1_structure, b1_structure, W2_structure, b2_structure,
           W1_virtual, b1_virtual, W2_virtual, b2_virtual,
           super_nodes):
    r = _E // 128
    eis = []
    for ei in (ei_text, ei_vision, ei_structure):
        eis.append(ei[0].reshape(r, 128))
        eis.append(ei[1].reshape(r, 128))
    b1s = [b.reshape(1, _H) for b in (b1_text, b1_vision, b1_structure, b1_virtual)]
    b2s = [b.reshape(1, _S) for b in (b2_text, b2_vision, b2_structure, b2_virtual)]

    f2, eflat = pl.pallas_call(
        _encode_body,
        out_shape=(
            jax.ShapeDtypeStruct((_M, _S), jnp.float32),
            jax.ShapeDtypeStruct((_EROWS, 128), jnp.int32),
        ),
    )(x_text, x_vision, x_structure, super_nodes,
      W1_text, b1s[0], W2_text, b2s[0],
      W1_vision, b1s[1], W2_vision, b2s[1],
      W1_structure, b1s[2], W2_structure, b2s[2],
      W1_virtual, b1s[3], W2_virtual, b2s[3],
      *eis)

    rightL, rightR, bottom = pl.pallas_call(
        _strips_body,
        out_shape=(
            jax.ShapeDtypeStruct((_T, 128), jnp.float32),
            jax.ShapeDtypeStruct((_T, 128), jnp.float32),
            jax.ShapeDtypeStruct((_P, _M), jnp.float32),
        ),
    )(f2)
    bottom_flat = bottom.reshape(_P * _M)

    zeros2d = pl.pallas_call(
        _paint_body,
        grid=(_NBLK,),
        in_specs=[],
        out_specs=pl.BlockSpec((_M * _M // _NBLK // 128, 128), lambda i: (i, 0)),
        out_shape=jax.ShapeDtypeStruct((_M * _M // 128, 128), jnp.float32),
    )()

    ones = jnp.ones((128,), jnp.float32)
    out_ref = jax.new_ref(zeros2d.reshape(_M * _M))
    _get_scatter()(eflat, ones, rightL, rightR, bottom_flat, out_ref)
    flat = jax.freeze(out_ref).reshape(_M * _M // 128, 128)
    return pl.pallas_call(
        _relayout_body,
        grid=(_NBLK,),
        in_specs=[pl.BlockSpec((_M * _M // _NBLK // 128, 128), lambda i: (i, 0))],
        out_specs=pl.BlockSpec((_RB, _M), lambda i: (i, 0)),
        out_shape=jax.ShapeDtypeStruct((_M, _M), jnp.float32),
    )(flat)


# speedup vs baseline: 1.0026x; 1.0026x over previous
import functools

import jax
import jax.numpy as jnp
from jax import lax
from jax.experimental import pallas as pl
from jax.experimental.pallas import tpu as pltpu
from jax.experimental.pallas import tpu_sc as plsc

_N = 2048
_D = 512
_H = 256
_S = 128
_P = 256
_E = 65536
_T = 3 * _N
_M = _T + _P
_RB = 256
_NBLK = _M // _RB

_NC, _NS = 2, 16
_NW = _NC * _NS
_EROWS = 3 * _E // 128
_WROWS = _EROWS // _NW
_RROWS = _T // _NW
_BROWS = _P // _NW

_PREC = jax.lax.Precision.HIGHEST


def _mlp(x, W1, b1, W2, b2):
    h = jnp.maximum(jnp.dot(x, W1.T, precision=_PREC) + b1, 0.0)
    return jnp.dot(h, W2.T, precision=_PREC) + b2


def _encode_body(xt, xv, xs, sp,
                 W1t, b1t, W2t, b2t,
                 W1v, b1v, W2v, b2v,
                 W1s, b1s, W2s, b2s,
                 W1u, b1u, W2u, b2u,
                 e0t, e1t, e0v, e1v, e0s, e1s,
                 ef_ref, rl_ref, rr_ref, bot_ref):
    ft = _mlp(xt[...], W1t[...], b1t[...], W2t[...], b2t[...])
    fv = _mlp(xv[...], W1v[...], b1v[...], W2v[...], b2v[...])
    fs = _mlp(xs[...], W1s[...], b1s[...], W2s[...], b2s[...])
    sn = _mlp(sp[...], W1u[...], b1u[...], W2u[...], b2u[...])
    f2 = jnp.concatenate([ft, fv, fs, sn], axis=0)
    r = _E // 128
    ef_ref[0:r, :] = e0t[...] * _M + e1t[...]
    ef_ref[r:2 * r, :] = e0v[...] * _M + e1v[...] + _N * (_M + 1)
    ef_ref[2 * r:3 * r, :] = e0s[...] * _M + e1s[...] + 2 * _N * (_M + 1)
    right = jax.nn.sigmoid(
        lax.dot_general(f2[:_T], sn, (((1,), (1,)), ((), ())), precision=_PREC))
    rl_ref[...] = right[:, :128]
    rr_ref[...] = right[:, 128:]
    bot_ref[...] = jax.nn.sigmoid(
        lax.dot_general(sn, f2, (((1,), (1,)), ((), ())), precision=_PREC))


def _relayout_body(in_ref, out_ref):
    out_ref[...] = in_ref[...].reshape(_RB, _M)


def _paint_body(out_ref):
    out_ref[...] = jnp.zeros((_M * _M // _NBLK // 128, 128), jnp.float32)


@functools.cache
def _get_scatter():
    mesh = plsc.VectorSubcoreMesh(
        core_axis_name="c", subcore_axis_name="s",
        num_cores=_NC, num_subcores=_NS)

    @functools.partial(
        pl.kernel,
        out_type=(),
        mesh=mesh,
        scratch_types=[
            pltpu.VMEM((_WROWS, 128), jnp.int32),
            pltpu.VMEM((128,), jnp.float32),
            pltpu.VMEM((_RROWS, 256), jnp.float32),
            pltpu.VMEM((_BROWS * _M,), jnp.float32),
            pltpu.SemaphoreType.DMA,
            pltpu.SemaphoreType.DMA,
        ],
    )
    def _scatter(ef_hbm, ones_hbm, rl_hbm, rr_hbm, bot_hbm, out_ref,
                 idx_v, ones_v, right_v, bot_v, sem, sem2):
        wid = lax.axis_index("s") * _NC + lax.axis_index("c")
        base = wid * _WROWS
        pltpu.sync_copy(ef_hbm.at[pl.ds(base, _WROWS)], idx_v)
        pltpu.sync_copy(ones_hbm, ones_v)
        descs = [
            pltpu.async_copy(ones_v, out_ref.at[idx_v.at[j]], sem)
            for j in range(_WROWS)
        ]
        rbase = wid * _RROWS
        pltpu.sync_copy(rl_hbm.at[pl.ds(rbase, _RROWS)], right_v.at[:, pl.ds(0, 128)])
        pltpu.sync_copy(rr_hbm.at[pl.ds(rbase, _RROWS)], right_v.at[:, pl.ds(128, 128)])

        def _fire(k, _):
            pltpu.async_copy(
                right_v.at[k],
                out_ref.at[pl.ds((rbase + k) * _M + _T, 256)],
                sem2)
            return _
        lax.fori_loop(0, _RROWS, _fire, 0)
        bbase = _T * _M + wid * _BROWS * _M
        pltpu.sync_copy(bot_hbm.at[pl.ds(wid * _BROWS * _M, _BROWS * _M)], bot_v)
        pltpu.sync_copy(bot_v, out_ref.at[pl.ds(bbase, _BROWS * _M)])

        def _drain(k, _):
            pltpu.make_async_copy(
                right_v.at[0],
                out_ref.at[pl.ds(_T, 256)],
                sem2).wait()
            return _
        lax.fori_loop(0, _RROWS, _drain, 0)
        for d in descs:
            d.wait()

    return _scatter


def kernel(x_text, x_vision, x_structure, ei_text, ei_vision, ei_structure,
           W1_text, b1_text, W2_text, b2_text,
           W1_vision, b1_vision, W2_vision, b2_vision,
           W1_structure, b1_structure, W2_structure, b2_structure,
           W1_virtual, b1_virtual, W2_virtual, b2_virtual,
           super_nodes):
    r = _E // 128
    eis = []
    for ei in (ei_text, ei_vision, ei_structure):
        eis.append(ei[0].reshape(r, 128))
        eis.append(ei[1].reshape(r, 128))
    b1s = [b.reshape(1, _H) for b in (b1_text, b1_vision, b1_structure, b1_virtual)]
    b2s = [b.reshape(1, _S) for b in (b2_text, b2_vision, b2_structure, b2_virtual)]

    eflat, rightL, rightR, bottom = pl.pallas_call(
        _encode_body,
        out_shape=(
            jax.ShapeDtypeStruct((_EROWS, 128), jnp.int32),
            jax.ShapeDtypeStruct((_T, 128), jnp.float32),
            jax.ShapeDtypeStruct((_T, 128), jnp.float32),
            jax.ShapeDtypeStruct((_P, _M), jnp.float32),
        ),
    )(x_text, x_vision, x_structure, super_nodes,
      W1_text, b1s[0], W2_text, b2s[0],
      W1_vision, b1s[1], W2_vision, b2s[1],
      W1_structure, b1s[2], W2_structure, b2s[2],
      W1_virtual, b1s[3], W2_virtual, b2s[3],
      *eis)
    bottom_flat = bottom.reshape(_P * _M)

    zeros2d = pl.pallas_call(
        _paint_body,
        grid=(_NBLK,),
        in_specs=[],
        out_specs=pl.BlockSpec((_M * _M // _NBLK // 128, 128), lambda i: (i, 0)),
        out_shape=jax.ShapeDtypeStruct((_M * _M // 128, 128), jnp.float32),
    )()

    ones = jnp.ones((128,), jnp.float32)
    out_ref = jax.new_ref(zeros2d.reshape(_M * _M))
    _get_scatter()(eflat, ones, rightL, rightR, bottom_flat, out_ref)
    flat = jax.freeze(out_ref).reshape(_M * _M // 128, 128)
    return pl.pallas_call(
        _relayout_body,
        grid=(_NBLK,),
        in_specs=[pl.BlockSpec((_M * _M // _NBLK // 128, 128), lambda i: (i, 0))],
        out_specs=pl.BlockSpec((_RB, _M), lambda i: (i, 0)),
        out_shape=jax.ShapeDtypeStruct((_M, _M), jnp.float32),
    )(flat)


# speedup vs baseline: 1.0606x; 1.0579x over previous
import functools

import jax
import jax.numpy as jnp
from jax import lax
from jax.experimental import pallas as pl
from jax.experimental.pallas import tpu as pltpu
from jax.experimental.pallas import tpu_sc as plsc

_N = 2048
_D = 512
_H = 256
_S = 128
_P = 256
_E = 65536
_T = 3 * _N
_M = _T + _P
_RB = 256
_L = _M // 128

_NC, _NS = 2, 16
_NW = _NC * _NS
_WROWS = (_E // 128) // _NW
_RROWS = _N // _NW
_BROWS = _P // _NW

_PREC = jax.lax.Precision.HIGHEST


def _mlp(x, W1, b1, W2, b2):
    h = jnp.maximum(jnp.dot(x, W1.T, precision=_PREC) + b1, 0.0)
    return h @ W2.T + b2


def _encode_body(xt, xv, xs, sp,
                 W1t, b1t, W2t, b2t,
                 W1v, b1v, W2v, b2v,
                 W1s, b1s, W2s, b2s,
                 W1u, b1u, W2u, b2u,
                 e0t, e1t, e0v, e1v, e0s, e1s,
                 ef0_ref, ef1_ref, ef2_ref, rl_ref, rr_ref, bot_ref):
    ft = _mlp(xt[...], W1t[...], b1t[...], W2t[...], b2t[...])
    fv = _mlp(xv[...], W1v[...], b1v[...], W2v[...], b2v[...])
    fs = _mlp(xs[...], W1s[...], b1s[...], W2s[...], b2s[...])
    sn = _mlp(sp[...], W1u[...], b1u[...], W2u[...], b2u[...])
    f2 = jnp.concatenate([ft, fv, fs, sn], axis=0)
    ef0_ref[...] = e0t[...] * _M + e1t[...]
    ef1_ref[...] = e0v[...] * _M + e1v[...] + _N
    ef2_ref[...] = e0s[...] * _M + e1s[...] + 2 * _N
    right = jax.nn.sigmoid(
        lax.dot_general(f2[:_T], sn, (((1,), (1,)), ((), ())), precision=_PREC))
    rl_ref[...] = right[:, :128]
    rr_ref[...] = right[:, 128:]
    bot_ref[...] = jax.nn.sigmoid(
        lax.dot_general(sn, f2, (((1,), (1,)), ((), ())), precision=_PREC))


def _zeros_body(out_ref):
    out_ref[...] = jnp.zeros((_RB * _L, 128), jnp.float32)


def _relayout_body(in_ref, out_ref):
    out_ref[...] = in_ref[...].reshape(_RB, _M)


def _zeros(nrows):
    nblk = nrows // _RB
    return pl.pallas_call(
        _zeros_body,
        grid=(nblk,),
        in_specs=[],
        out_specs=pl.BlockSpec((_RB * _L, 128), lambda i: (i, 0)),
        out_shape=jax.ShapeDtypeStruct((nrows * _L, 128), jnp.float32),
    )()


def _relayout(band2d, prev, nblk, blk0):
    if prev is None:
        return pl.pallas_call(
            _relayout_body,
            grid=(nblk,),
            in_specs=[pl.BlockSpec((_RB * _L, 128), lambda i: (i, 0))],
            out_specs=pl.BlockSpec((_RB, _M), lambda i: (i + blk0, 0)),
            out_shape=jax.ShapeDtypeStruct((_M, _M), jnp.float32),
        )(band2d)

    def body(in_ref, _f_ref, out_ref):
        _relayout_body(in_ref, out_ref)

    return pl.pallas_call(
        body,
        grid=(nblk,),
        in_specs=[pl.BlockSpec((_RB * _L, 128), lambda i: (i, 0)),
                  pl.BlockSpec(memory_space=pltpu.MemorySpace.HBM)],
        out_specs=pl.BlockSpec((_RB, _M), lambda i: (i + blk0, 0)),
        out_shape=jax.ShapeDtypeStruct((_M, _M), jnp.float32),
        input_output_aliases={1: 0},
    )(band2d, prev)


@functools.cache
def _get_scatter(with_bottom):
    mesh = plsc.VectorSubcoreMesh(
        core_axis_name="c", subcore_axis_name="s",
        num_cores=_NC, num_subcores=_NS)

    scratch = [
        pltpu.VMEM((_WROWS, 128), jnp.int32),
        pltpu.VMEM((128,), jnp.float32),
        pltpu.VMEM((_RROWS, 256), jnp.float32),
        pltpu.SemaphoreType.DMA,
        pltpu.SemaphoreType.DMA,
    ]
    if with_bottom:
        scratch.insert(3, pltpu.VMEM((_BROWS * _M,), jnp.float32))

    def _body(ef_hbm, ones_hbm, rl_hbm, rr_hbm, bot_hbm, out_ref,
              idx_v, ones_v, right_v, bot_v, sem, sem2):
        wid = lax.axis_index("s") * _NC + lax.axis_index("c")
        pltpu.sync_copy(ef_hbm.at[pl.ds(wid * _WROWS, _WROWS)], idx_v)
        pltpu.sync_copy(ones_hbm, ones_v)
        descs = [
            pltpu.async_copy(ones_v, out_ref.at[idx_v.at[j]], sem)
            for j in range(_WROWS)
        ]
        rbase = wid * _RROWS
        pltpu.sync_copy(rl_hbm.at[pl.ds(rbase, _RROWS)],
                        right_v.at[:, pl.ds(0, 128)])
        pltpu.sync_copy(rr_hbm.at[pl.ds(rbase, _RROWS)],
                        right_v.at[:, pl.ds(128, 128)])

        def _fire(k, c):
            pltpu.async_copy(
                right_v.at[k],
                out_ref.at[pl.ds((rbase + k) * _M + _T, 256)],
                sem2)
            return c
        lax.fori_loop(0, _RROWS, _fire, 0)
        if with_bottom:
            bbase = _N * _M + wid * _BROWS * _M
            pltpu.sync_copy(bot_hbm.at[pl.ds(wid * _BROWS * _M, _BROWS * _M)],
                            bot_v)
            pltpu.sync_copy(bot_v, out_ref.at[pl.ds(bbase, _BROWS * _M)])

        def _drain(k, c):
            pltpu.make_async_copy(
                right_v.at[0],
                out_ref.at[pl.ds(_T, 256)],
                sem2).wait()
            return c
        lax.fori_loop(0, _RROWS, _drain, 0)
        for d in descs:
            d.wait()

    if with_bottom:
        def _scatter(ef, ones, rl, rr, bot, out_ref, idx, onev, rv, bv, s1, s2):
            _body(ef, ones, rl, rr, bot, out_ref, idx, onev, rv, bv, s1, s2)
    else:
        def _scatter(ef, ones, rl, rr, out_ref, idx, onev, rv, s1, s2):
            _body(ef, ones, rl, rr, None, out_ref, idx, onev, rv, None, s1, s2)

    return functools.partial(
        pl.kernel, out_type=(), mesh=mesh, scratch_types=scratch)(_scatter)


def kernel(x_text, x_vision, x_structure, ei_text, ei_vision, ei_structure,
           W1_text, b1_text, W2_text, b2_text,
           W1_vision, b1_vision, W2_vision, b2_vision,
           W1_structure, b1_structure, W2_structure, b2_structure,
           W1_virtual, b1_virtual, W2_virtual, b2_virtual,
           super_nodes):
    r = _E // 128
    eis = []
    for ei in (ei_text, ei_vision, ei_structure):
        eis.append(ei[0].reshape(r, 128))
        eis.append(ei[1].reshape(r, 128))
    b1s = [b.reshape(1, _H) for b in (b1_text, b1_vision, b1_structure, b1_virtual)]
    b2s = [b.reshape(1, _S) for b in (b2_text, b2_vision, b2_structure, b2_virtual)]

    ef0, ef1, ef2, rightL, rightR, bottom = pl.pallas_call(
        _encode_body,
        out_shape=(
            jax.ShapeDtypeStruct((r, 128), jnp.int32),
            jax.ShapeDtypeStruct((r, 128), jnp.int32),
            jax.ShapeDtypeStruct((r, 128), jnp.int32),
            jax.ShapeDtypeStruct((_T, 128), jnp.float32),
            jax.ShapeDtypeStruct((_T, 128), jnp.float32),
            jax.ShapeDtypeStruct((_P, _M), jnp.float32),
        ),
    )(x_text, x_vision, x_structure, super_nodes,
      W1_text, b1s[0], W2_text, b2s[0],
      W1_vision, b1s[1], W2_vision, b2s[1],
      W1_structure, b1s[2], W2_structure, b2s[2],
      W1_virtual, b1s[3], W2_virtual, b2s[3],
      *eis)
    bottom_flat = bottom.reshape(_P * _M)
    ones = jnp.ones((128,), jnp.float32)

    efs = (ef0, ef1, ef2)
    bands = []
    for m in range(3):
        nrows = _N if m < 2 else _N + _P
        zm = _zeros(nrows)
        ref_m = jax.new_ref(zm.reshape(nrows * _M))
        rl_m = lax.slice_in_dim(rightL, m * _N, (m + 1) * _N, axis=0)
        rr_m = lax.slice_in_dim(rightR, m * _N, (m + 1) * _N, axis=0)
        if m < 2:
            _get_scatter(False)(efs[m], ones, rl_m, rr_m, ref_m)
        else:
            _get_scatter(True)(efs[m], ones, rl_m, rr_m, bottom_flat, ref_m)
        bands.append(jax.freeze(ref_m).reshape(nrows * _L, 128))

    out = _relayout(bands[0], None, _N // _RB, 0)
    out = _relayout(bands[1], out, _N // _RB, 8)
    out = _relayout(bands[2], out, (_N + _P) // _RB, 16)
    return out


# speedup vs baseline: 1.0628x; 1.0021x over previous
import functools

import jax
import jax.numpy as jnp
from jax import lax
from jax.experimental import pallas as pl
from jax.experimental.pallas import tpu as pltpu
from jax.experimental.pallas import tpu_sc as plsc

_N = 2048
_D = 512
_H = 256
_S = 128
_P = 256
_E = 65536
_T = 3 * _N
_M = _T + _P
_RB = 256
_L = _M // 128

_NC, _NS = 2, 16
_NW = _NC * _NS
_WROWS = (_E // 128) // _NW
_RROWS = _N // _NW
_BROWS = _P // _NW

_PREC = jax.lax.Precision.HIGHEST


def _mlp(x, W1, b1, W2, b2):
    h = jnp.maximum(jnp.dot(x, W1.T, precision=_PREC) + b1, 0.0)
    return h @ W2.T + b2


def _encode_body(xt, xv, xs, sp,
                 W1t, b1t, W2t, b2t,
                 W1v, b1v, W2v, b2v,
                 W1s, b1s, W2s, b2s,
                 W1u, b1u, W2u, b2u,
                 e0t, e1t, e0v, e1v, e0s, e1s,
                 ef0_ref, ef1_ref, rl_ref, rr_ref, bot_ref):
    ft = _mlp(xt[...], W1t[...], b1t[...], W2t[...], b2t[...])
    fv = _mlp(xv[...], W1v[...], b1v[...], W2v[...], b2v[...])
    fs = _mlp(xs[...], W1s[...], b1s[...], W2s[...], b2s[...])
    sn = _mlp(sp[...], W1u[...], b1u[...], W2u[...], b2u[...])
    f2 = jnp.concatenate([ft, fv, fs, sn], axis=0)
    r = _E // 128
    ef0_ref[0:r, :] = e0t[...] * _M + e1t[...]
    ef0_ref[r:2 * r, :] = e0v[...] * _M + e1v[...] + _N * (_M + 1)
    ef1_ref[...] = e0s[...] * _M + e1s[...] + 2 * _N
    right = jax.nn.sigmoid(
        lax.dot_general(f2[:_T], sn, (((1,), (1,)), ((), ())), precision=_PREC))
    rl_ref[...] = right[:, :128]
    rr_ref[...] = right[:, 128:]
    bot_ref[...] = jax.nn.sigmoid(
        lax.dot_general(sn, f2, (((1,), (1,)), ((), ())), precision=_PREC))


def _zeros_body(out_ref):
    out_ref[...] = jnp.zeros((_RB * _L, 128), jnp.float32)


def _relayout_body(in_ref, out_ref):
    out_ref[...] = in_ref[...].reshape(_RB, _M)


def _zeros(nrows):
    nblk = nrows // _RB
    return pl.pallas_call(
        _zeros_body,
        grid=(nblk,),
        in_specs=[],
        out_specs=pl.BlockSpec((_RB * _L, 128), lambda i: (i, 0)),
        out_shape=jax.ShapeDtypeStruct((nrows * _L, 128), jnp.float32),
    )()


def _relayout(band2d, prev, nblk, blk0):
    if prev is None:
        return pl.pallas_call(
            _relayout_body,
            grid=(nblk,),
            in_specs=[pl.BlockSpec((_RB * _L, 128), lambda i: (i, 0))],
            out_specs=pl.BlockSpec((_RB, _M), lambda i: (i + blk0, 0)),
            out_shape=jax.ShapeDtypeStruct((_M, _M), jnp.float32),
        )(band2d)

    def body(in_ref, _f_ref, out_ref):
        _relayout_body(in_ref, out_ref)

    return pl.pallas_call(
        body,
        grid=(nblk,),
        in_specs=[pl.BlockSpec((_RB * _L, 128), lambda i: (i, 0)),
                  pl.BlockSpec(memory_space=pltpu.MemorySpace.HBM)],
        out_specs=pl.BlockSpec((_RB, _M), lambda i: (i + blk0, 0)),
        out_shape=jax.ShapeDtypeStruct((_M, _M), jnp.float32),
        input_output_aliases={1: 0},
    )(band2d, prev)


@functools.cache
def _get_scatter(wrows, rrows, with_bottom):
    mesh = plsc.VectorSubcoreMesh(
        core_axis_name="c", subcore_axis_name="s",
        num_cores=_NC, num_subcores=_NS)

    scratch = [
        pltpu.VMEM((wrows, 128), jnp.int32),
        pltpu.VMEM((128,), jnp.float32),
        pltpu.VMEM((rrows, 256), jnp.float32),
        pltpu.SemaphoreType.DMA,
        pltpu.SemaphoreType.DMA,
    ]
    if with_bottom:
        scratch.insert(3, pltpu.VMEM((_BROWS * _M,), jnp.float32))

    def _body(ef_hbm, ones_hbm, rl_hbm, rr_hbm, bot_hbm, out_ref,
              idx_v, ones_v, right_v, bot_v, sem, sem2):
        wid = lax.axis_index("s") * _NC + lax.axis_index("c")
        pltpu.sync_copy(ef_hbm.at[pl.ds(wid * wrows, wrows)], idx_v)
        pltpu.sync_copy(ones_hbm, ones_v)
        descs = [
            pltpu.async_copy(ones_v, out_ref.at[idx_v.at[j]], sem)
            for j in range(wrows)
        ]
        rbase = wid * rrows
        pltpu.sync_copy(rl_hbm.at[pl.ds(rbase, rrows)],
                        right_v.at[:, pl.ds(0, 128)])
        pltpu.sync_copy(rr_hbm.at[pl.ds(rbase, rrows)],
                        right_v.at[:, pl.ds(128, 128)])

        def _fire(k, c):
            pltpu.async_copy(
                right_v.at[k],
                out_ref.at[pl.ds((rbase + k) * _M + _T, 256)],
                sem2)
            return c
        lax.fori_loop(0, rrows, _fire, 0)
        if with_bottom:
            bbase = _N * _M + wid * _BROWS * _M
            pltpu.sync_copy(bot_hbm.at[pl.ds(wid * _BROWS * _M, _BROWS * _M)],
                            bot_v)
            pltpu.sync_copy(bot_v, out_ref.at[pl.ds(bbase, _BROWS * _M)])

        def _drain(k, c):
            pltpu.make_async_copy(
                right_v.at[0],
                out_ref.at[pl.ds(_T, 256)],
                sem2).wait()
            return c
        lax.fori_loop(0, rrows, _drain, 0)
        for d in descs:
            d.wait()

    if with_bottom:
        def _scatter(ef, ones, rl, rr, bot, out_ref, idx, onev, rv, bv, s1, s2):
            _body(ef, ones, rl, rr, bot, out_ref, idx, onev, rv, bv, s1, s2)
    else:
        def _scatter(ef, ones, rl, rr, out_ref, idx, onev, rv, s1, s2):
            _body(ef, ones, rl, rr, None, out_ref, idx, onev, rv, None, s1, s2)

    return functools.partial(
        pl.kernel, out_type=(), mesh=mesh, scratch_types=scratch)(_scatter)


def kernel(x_text, x_vision, x_structure, ei_text, ei_vision, ei_structure,
           W1_text, b1_text, W2_text, b2_text,
           W1_vision, b1_vision, W2_vision, b2_vision,
           W1_structure, b1_structure, W2_structure, b2_structure,
           W1_virtual, b1_virtual, W2_virtual, b2_virtual,
           super_nodes):
    r = _E // 128
    eis = []
    for ei in (ei_text, ei_vision, ei_structure):
        eis.append(ei[0].reshape(r, 128))
        eis.append(ei[1].reshape(r, 128))
    b1s = [b.reshape(1, _H) for b in (b1_text, b1_vision, b1_structure, b1_virtual)]
    b2s = [b.reshape(1, _S) for b in (b2_text, b2_vision, b2_structure, b2_virtual)]

    ef0, ef1, rightL, rightR, bottom = pl.pallas_call(
        _encode_body,
        out_shape=(
            jax.ShapeDtypeStruct((2 * r, 128), jnp.int32),
            jax.ShapeDtypeStruct((r, 128), jnp.int32),
            jax.ShapeDtypeStruct((_T, 128), jnp.float32),
            jax.ShapeDtypeStruct((_T, 128), jnp.float32),
            jax.ShapeDtypeStruct((_P, _M), jnp.float32),
        ),
    )(x_text, x_vision, x_structure, super_nodes,
      W1_text, b1s[0], W2_text, b2s[0],
      W1_vision, b1s[1], W2_vision, b2s[1],
      W1_structure, b1s[2], W2_structure, b2s[2],
      W1_virtual, b1s[3], W2_virtual, b2s[3],
      *eis)
    bottom_flat = bottom.reshape(_P * _M)
    ones = jnp.ones((128,), jnp.float32)

    n0, n1 = 2 * _N, _N + _P
    z0 = _zeros(n0)
    ref0 = jax.new_ref(z0.reshape(n0 * _M))
    _get_scatter(2 * _E // 128 // _NW, n0 // _NW, False)(
        ef0, ones,
        lax.slice_in_dim(rightL, 0, n0, axis=0),
        lax.slice_in_dim(rightR, 0, n0, axis=0),
        ref0)
    band0 = jax.freeze(ref0).reshape(n0 * _L, 128)

    z1 = _zeros(n1)
    ref1 = jax.new_ref(z1.reshape(n1 * _M))
    _get_scatter(_E // 128 // _NW, _N // _NW, True)(
        ef1, ones,
        lax.slice_in_dim(rightL, n0, _T, axis=0),
        lax.slice_in_dim(rightR, n0, _T, axis=0),
        bottom_flat, ref1)
    band1 = jax.freeze(ref1).reshape(n1 * _L, 128)

    out = _relayout(band0, None, n0 // _RB, 0)
    out = _relayout(band1, out, n1 // _RB, n0 // _RB)
    return out
